# baseline (speedup 1.0000x reference)
import jax
import jax.numpy as jnp
from jax import lax
from jax.experimental import pallas as pl

_N = 207
_NP = 208
_HID = 64
_IN = 2
_OUT = 1
_T = 12
_B = 16
_NM = 3
_TOPK = 30
_BNP = _B * _NP


def _body(xt_ref, wih_ref, whh_ref, bih_ref, bhh_ref, wkey_ref, wq_ref,
          e0gx_ref, e0bg_ref, e0cx_ref, e0bc_ref,
          e1gx_ref, e1bg_ref, e1cx_ref, e1bc_ref,
          d0gx_ref, d0bg_ref, d0cx_ref, d0bc_ref,
          d1gx_ref, d1bg_ref, d1cx_ref, d1bc_ref,
          projw_ref, projb_ref, out_ref):
    f32 = jnp.float32
    sig = jax.nn.sigmoid

    wih = wih_ref[...]
    whh = whh_ref[...]
    bih = bih_ref[...]
    bhh = bhh_ref[...]

    def gru_step(t, h):
        xtt = xt_ref[pl.ds(t, 1)].reshape(_BNP, _IN)
        gi = jnp.dot(xtt, wih, preferred_element_type=f32) + bih
        gh = jnp.dot(h, whh, preferred_element_type=f32) + bhh
        r = sig(gi[:, :_HID] + gh[:, :_HID])
        z = sig(gi[:, _HID:2 * _HID] + gh[:, _HID:2 * _HID])
        n = jnp.tanh(gi[:, 2 * _HID:] + r * gh[:, 2 * _HID:])
        return (1.0 - z) * n + z * h

    h = lax.fori_loop(0, _T, gru_step, jnp.zeros((_BNP, _HID), f32))

    keyv = jnp.dot(h, wkey_ref[...], preferred_element_type=f32)
    qryv = jnp.dot(h, wq_ref[...], preferred_element_type=f32)
    col = lax.broadcasted_iota(jnp.int32, (_NP, _NP), 1)
    colmask = col < _N
    acc = jnp.zeros((_NP, _NP), f32)
    for b in range(_B):
        kb = keyv[b * _NP:(b + 1) * _NP, :]
        qb = qryv[b * _NP:(b + 1) * _NP, :]
        ab = lax.dot_general(kb, qb, (((1,), (1,)), ((), ())),
                             preferred_element_type=f32)
        ab = jnp.maximum(ab, 0.0)
        ab = jnp.where(colmask, ab, -1e30)
        ab = ab - jnp.max(ab, axis=1, keepdims=True)
        e = jnp.where(colmask, jnp.exp(ab), 0.0)
        acc = acc + e / jnp.sum(e, axis=1, keepdims=True)
    adj = acc * (1.0 / _B)

    work = jnp.where(colmask, adj, -1.0)
    thresh = None
    for _ in range(_TOPK):
        thresh = jnp.max(work, axis=1, keepdims=True)
        ismax = work == thresh
        pos = jnp.min(jnp.where(ismax, col, _NP), axis=1, keepdims=True)
        work = jnp.where(col == pos, -1.0, work)
    rowmask = lax.broadcasted_iota(jnp.int32, (_NP, _NP), 0) < _N
    adj_k = jnp.where((adj >= thresh) & colmask & rowmask, adj, 0.0)
    d = jnp.sum(adj_k, axis=1, keepdims=True)
    dinv = jnp.where(d > 0.0, 1.0 / d, 0.0)
    support = dinv * adj_k

    def cell(xin, hin, wgf, bg, wcf, bc):
        outs = []
        for b in range(_B):
            xb = xin[b * _NP:(b + 1) * _NP, :]
            hb = hin[b * _NP:(b + 1) * _NP, :]
            c0 = jnp.concatenate([xb, hb], axis=1)
            c1 = jnp.dot(support, c0, preferred_element_type=f32)
            c2 = 2.0 * jnp.dot(support, c1, preferred_element_type=f32) - c0
            g = sig(jnp.dot(jnp.concatenate([c0, c1, c2], axis=1), wgf,
                            preferred_element_type=f32) + bg)
            r = g[:, :_HID]
            u = g[:, _HID:]
            rh = r * hb
            y0 = jnp.concatenate([xb, rh], axis=1)
            y1 = jnp.dot(support, y0, preferred_element_type=f32)
            y2 = 2.0 * jnp.dot(support, y1, preferred_element_type=f32) - y0
            cand = jnp.tanh(jnp.dot(jnp.concatenate([y0, y1, y2], axis=1), wcf,
                                    preferred_element_type=f32) + bc)
            outs.append(u * hb + (1.0 - u) * cand)
        return jnp.concatenate(outs, axis=0)

    e0g = e0gx_ref[...]; e0bg = e0bg_ref[...]
    e0c = e0cx_ref[...]; e0bc = e0bc_ref[...]
    e1g = e1gx_ref[...]; e1bg = e1bg_ref[...]
    e1c = e1cx_ref[...]; e1bc = e1bc_ref[...]
    d0g = d0gx_ref[...]; d0bg = d0bg_ref[...]
    d0c = d0cx_ref[...]; d0bc = d0bc_ref[...]
    d1g = d1gx_ref[...]; d1bg = d1bg_ref[...]
    d1c = d1cx_ref[...]; d1bc = d1bc_ref[...]
    projw = projw_ref[...]
    projb = projb_ref[...]

    def enc_step(t, hh):
        h0, h1 = hh
        x_t = xt_ref[pl.ds(t, 1)].reshape(_BNP, _IN)
        h0 = cell(x_t, h0, e0g, e0bg, e0c, e0bc)
        h1 = cell(h0, h1, e1g, e1bg, e1c, e1bc)
        return (h0, h1)

    zst = jnp.zeros((_BNP, _HID), f32)
    h0, h1 = lax.fori_loop(0, _T, enc_step, (zst, zst))

    tcol = lax.broadcasted_iota(jnp.int32, (_BNP, _T), 1)

    def dec_step(t, carry):
        g0, g1, cur, outacc = carry
        g0 = cell(cur, g0, d0g, d0bg, d0c, d0bc)
        g1 = cell(g0, g1, d1g, d1bg, d1c, d1bc)
        p = jnp.sum(g1 * projw, axis=1, keepdims=True) + projb
        outacc = jnp.where(tcol == t, p, outacc)
        return (g0, g1, p, outacc)

    _, _, _, outacc = lax.fori_loop(
        0, _T, dec_step,
        (h0, h1, jnp.zeros((_BNP, _OUT), f32), jnp.zeros((_BNP, _T), f32)))
    out_ref[...] = outacc


def kernel(x, enc0_Wg, enc0_bg, enc0_Wc, enc0_bc, enc1_Wg, enc1_bg, enc1_Wc, enc1_bc,
           dec0_Wg, dec0_bg, dec0_Wc, dec0_bc, dec1_Wg, dec1_bg, dec1_Wc, dec1_bc,
           proj_W, proj_b, gru_Wih, gru_Whh, gru_bih, gru_bhh, Wkey, Wquery):
    f32 = jnp.float32
    xt = jnp.transpose(x, (1, 0, 2, 3))
    xt = jnp.pad(xt, ((0, 0), (0, 0), (0, _NP - _N), (0, 0)))
    xt = xt.reshape(_T, _BNP, _IN)

    def deint(w):
        return jnp.concatenate([w[m::_NM] for m in range(_NM)], axis=0)

    args = (
        xt,
        gru_Wih.T.astype(f32), gru_Whh.T.astype(f32),
        gru_bih[None], gru_bhh[None],
        Wkey, Wquery,
        deint(enc0_Wg), enc0_bg[None], deint(enc0_Wc), enc0_bc[None],
        deint(enc1_Wg), enc1_bg[None], deint(enc1_Wc), enc1_bc[None],
        deint(dec0_Wg), dec0_bg[None], deint(dec0_Wc), dec0_bc[None],
        deint(dec1_Wg), dec1_bg[None], deint(dec1_Wc), dec1_bc[None],
        proj_W.T, proj_b.reshape(1, 1),
    )

    out = pl.pallas_call(
        _body,
        out_shape=jax.ShapeDtypeStruct((_BNP, _T), f32),
    )(*args)
    out = out.reshape(_B, _NP, _T)
    return jnp.transpose(out, (0, 2, 1))[:, :, :_N, None]


# speedup vs baseline: 1.6780x; 1.6780x over previous
import jax
import jax.numpy as jnp
from jax import lax
from jax.experimental import pallas as pl

_N = 207
_NP = 208
_HID = 64
_IN = 2
_OUT = 1
_T = 12
_B = 16
_NM = 3
_TOPK = 30
_BNP = _B * _NP


def _body(xt_ref, wih_ref, whh_ref, bih_ref, bhh_ref, wkey_ref, wq_ref,
          e0gx_ref, e0bg_ref, e0cx_ref, e0bc_ref,
          e1gx_ref, e1bg_ref, e1cx_ref, e1bc_ref,
          d0gx_ref, d0bg_ref, d0cx_ref, d0bc_ref,
          d1gx_ref, d1bg_ref, d1cx_ref, d1bc_ref,
          projw_ref, projb_ref, out_ref):
    f32 = jnp.float32
    sig = jax.nn.sigmoid

    wih = wih_ref[...]
    whh = whh_ref[...]
    bih = bih_ref[...]
    bhh = bhh_ref[...]

    def gru_step(t, h):
        xtt = xt_ref[pl.ds(t, 1)].reshape(_BNP, _IN)
        gi = jnp.dot(xtt, wih, preferred_element_type=f32) + bih
        gh = jnp.dot(h, whh, preferred_element_type=f32) + bhh
        r = sig(gi[:, :_HID] + gh[:, :_HID])
        z = sig(gi[:, _HID:2 * _HID] + gh[:, _HID:2 * _HID])
        n = jnp.tanh(gi[:, 2 * _HID:] + r * gh[:, 2 * _HID:])
        return (1.0 - z) * n + z * h

    h = lax.fori_loop(0, _T, gru_step, jnp.zeros((_BNP, _HID), f32))

    keyv = jnp.dot(h, wkey_ref[...], preferred_element_type=f32)
    qryv = jnp.dot(h, wq_ref[...], preferred_element_type=f32)
    col = lax.broadcasted_iota(jnp.int32, (_NP, _NP), 1)
    colmask = col < _N
    acc = jnp.zeros((_NP, _NP), f32)
    for b in range(_B):
        kb = keyv[b * _NP:(b + 1) * _NP, :]
        qb = qryv[b * _NP:(b + 1) * _NP, :]
        ab = lax.dot_general(kb, qb, (((1,), (1,)), ((), ())),
                             preferred_element_type=f32)
        ab = jnp.maximum(ab, 0.0)
        ab = jnp.where(colmask, ab, -1e30)
        ab = ab - jnp.max(ab, axis=1, keepdims=True)
        e = jnp.where(colmask, jnp.exp(ab), 0.0)
        acc = acc + e / jnp.sum(e, axis=1, keepdims=True)
    adj = acc * (1.0 / _B)

    work = jnp.where(colmask, adj, -1.0)
    thresh = None
    for _ in range(_TOPK):
        thresh = jnp.max(work, axis=1, keepdims=True)
        ismax = work == thresh
        pos = jnp.min(jnp.where(ismax, col, _NP), axis=1, keepdims=True)
        work = jnp.where(col == pos, -1.0, work)
    rowmask = lax.broadcasted_iota(jnp.int32, (_NP, _NP), 0) < _N
    adj_k = jnp.where((adj >= thresh) & colmask & rowmask, adj, 0.0)
    d = jnp.sum(adj_k, axis=1, keepdims=True)
    dinv = jnp.where(d > 0.0, 1.0 / d, 0.0)
    support = dinv * adj_k

    def smul(xv):
        w = xv.shape[1]
        g = min(_B, max(1, 128 // w))
        parts = [None] * _B
        for gi in range(_B // g):
            xs = [xv[(gi * g + j) * _NP:(gi * g + j + 1) * _NP, :]
                  for j in range(g)]
            blk = xs[0] if g == 1 else jnp.concatenate(xs, axis=1)
            pr = jnp.dot(support, blk, preferred_element_type=f32)
            for j in range(g):
                parts[gi * g + j] = pr[:, j * w:(j + 1) * w]
        return jnp.concatenate(parts, axis=0)

    def cell(xin, hin, wgf, bg, wcf, bc):
        cin = xin.shape[1]
        c = cin + _HID
        wgx = jnp.concatenate([wgf[m * c:m * c + cin] for m in range(_NM)], axis=0)
        wgh = [wgf[m * c + cin:(m + 1) * c] for m in range(_NM)]
        wcx = jnp.concatenate([wcf[m * c:m * c + cin] for m in range(_NM)], axis=0)
        wch = [wcf[m * c + cin:(m + 1) * c] for m in range(_NM)]
        sx1 = smul(xin)
        sx2 = 2.0 * smul(sx1) - xin
        sh1 = smul(hin)
        sh2 = 2.0 * smul(sh1) - hin
        xstack = jnp.concatenate([xin, sx1, sx2], axis=1)
        g = sig(jnp.dot(xstack, wgx, preferred_element_type=f32)
                + jnp.dot(hin, wgh[0], preferred_element_type=f32)
                + jnp.dot(sh1, wgh[1], preferred_element_type=f32)
                + jnp.dot(sh2, wgh[2], preferred_element_type=f32) + bg)
        r = g[:, :_HID]
        u = g[:, _HID:]
        rh = r * hin
        t1 = smul(rh)
        t2 = 2.0 * smul(t1) - rh
        cand = jnp.tanh(jnp.dot(xstack, wcx, preferred_element_type=f32)
                        + jnp.dot(rh, wch[0], preferred_element_type=f32)
                        + jnp.dot(t1, wch[1], preferred_element_type=f32)
                        + jnp.dot(t2, wch[2], preferred_element_type=f32) + bc)
        return u * hin + (1.0 - u) * cand

    e0g = e0gx_ref[...]; e0bg = e0bg_ref[...]
    e0c = e0cx_ref[...]; e0bc = e0bc_ref[...]
    e1g = e1gx_ref[...]; e1bg = e1bg_ref[...]
    e1c = e1cx_ref[...]; e1bc = e1bc_ref[...]
    d0g = d0gx_ref[...]; d0bg = d0bg_ref[...]
    d0c = d0cx_ref[...]; d0bc = d0bc_ref[...]
    d1g = d1gx_ref[...]; d1bg = d1bg_ref[...]
    d1c = d1cx_ref[...]; d1bc = d1bc_ref[...]
    projw = projw_ref[...]
    projb = projb_ref[...]

    def enc_step(t, hh):
        h0, h1 = hh
        x_t = xt_ref[pl.ds(t, 1)].reshape(_BNP, _IN)
        h0 = cell(x_t, h0, e0g, e0bg, e0c, e0bc)
        h1 = cell(h0, h1, e1g, e1bg, e1c, e1bc)
        return (h0, h1)

    zst = jnp.zeros((_BNP, _HID), f32)
    h0, h1 = lax.fori_loop(0, _T, enc_step, (zst, zst))

    tcol = lax.broadcasted_iota(jnp.int32, (_BNP, _T), 1)

    def dec_step(t, carry):
        g0, g1, cur, outacc = carry
        g0 = cell(cur, g0, d0g, d0bg, d0c, d0bc)
        g1 = cell(g0, g1, d1g, d1bg, d1c, d1bc)
        p = jnp.sum(g1 * projw, axis=1, keepdims=True) + projb
        outacc = jnp.where(tcol == t, p, outacc)
        return (g0, g1, p, outacc)

    _, _, _, outacc = lax.fori_loop(
        0, _T, dec_step,
        (h0, h1, jnp.zeros((_BNP, _OUT), f32), jnp.zeros((_BNP, _T), f32)))
    out_ref[...] = outacc


def kernel(x, enc0_Wg, enc0_bg, enc0_Wc, enc0_bc, enc1_Wg, enc1_bg, enc1_Wc, enc1_bc,
           dec0_Wg, dec0_bg, dec0_Wc, dec0_bc, dec1_Wg, dec1_bg, dec1_Wc, dec1_bc,
           proj_W, proj_b, gru_Wih, gru_Whh, gru_bih, gru_bhh, Wkey, Wquery):
    f32 = jnp.float32
    xt = jnp.transpose(x, (1, 0, 2, 3))
    xt = jnp.pad(xt, ((0, 0), (0, 0), (0, _NP - _N), (0, 0)))
    xt = xt.reshape(_T, _BNP, _IN)

    def deint(w):
        return jnp.concatenate([w[m::_NM] for m in range(_NM)], axis=0)

    args = (
        xt,
        gru_Wih.T.astype(f32), gru_Whh.T.astype(f32),
        gru_bih[None], gru_bhh[None],
        Wkey, Wquery,
        deint(enc0_Wg), enc0_bg[None], deint(enc0_Wc), enc0_bc[None],
        deint(enc1_Wg), enc1_bg[None], deint(enc1_Wc), enc1_bc[None],
        deint(dec0_Wg), dec0_bg[None], deint(dec0_Wc), dec0_bc[None],
        deint(dec1_Wg), dec1_bg[None], deint(dec1_Wc), dec1_bc[None],
        proj_W.T, proj_b.reshape(1, 1),
    )

    out = pl.pallas_call(
        _body,
        out_shape=jax.ShapeDtypeStruct((_BNP, _T), f32),
    )(*args)
    out = out.reshape(_B, _NP, _T)
    return jnp.transpose(out, (0, 2, 1))[:, :, :_N, None]


# speedup vs baseline: 2.1301x; 1.2694x over previous
import jax
import jax.numpy as jnp
from jax import lax
from jax.experimental import pallas as pl

_N = 207
_NP = 208
_HID = 64
_IN = 2
_OUT = 1
_T = 12
_B = 16
_NM = 3
_TOPK = 30
_BNP = _B * _NP


def _body(xt_ref, wih_ref, whh_ref, bih_ref, bhh_ref, wkey_ref, wq_ref,
          e0gx_ref, e0bg_ref, e0cx_ref, e0bc_ref,
          e1gx_ref, e1bg_ref, e1cx_ref, e1bc_ref,
          d0gx_ref, d0bg_ref, d0cx_ref, d0bc_ref,
          d1gx_ref, d1bg_ref, d1cx_ref, d1bc_ref,
          projw_ref, projb_ref, out_ref):
    f32 = jnp.float32
    sig = jax.nn.sigmoid

    wih = wih_ref[...]
    whh = whh_ref[...]
    bih = bih_ref[...]
    bhh = bhh_ref[...]

    def gru_step(t, h):
        xtt = xt_ref[pl.ds(t, 1)].reshape(_BNP, _IN)
        gi = jnp.dot(xtt, wih, preferred_element_type=f32) + bih
        gh = jnp.dot(h, whh, preferred_element_type=f32) + bhh
        r = sig(gi[:, :_HID] + gh[:, :_HID])
        z = sig(gi[:, _HID:2 * _HID] + gh[:, _HID:2 * _HID])
        n = jnp.tanh(gi[:, 2 * _HID:] + r * gh[:, 2 * _HID:])
        return (1.0 - z) * n + z * h

    h = lax.fori_loop(0, _T, gru_step, jnp.zeros((_BNP, _HID), f32))

    keyv = jnp.dot(h, wkey_ref[...], preferred_element_type=f32)
    qryv = jnp.dot(h, wq_ref[...], preferred_element_type=f32)
    col = lax.broadcasted_iota(jnp.int32, (_NP, _NP), 1)
    colmask = col < _N
    acc = jnp.zeros((_NP, _NP), f32)
    for b in range(_B):
        kb = keyv[b * _NP:(b + 1) * _NP, :]
        qb = qryv[b * _NP:(b + 1) * _NP, :]
        ab = lax.dot_general(kb, qb, (((1,), (1,)), ((), ())),
                             preferred_element_type=f32)
        ab = jnp.maximum(ab, 0.0)
        ab = jnp.where(colmask, ab, -1e30)
        ab = ab - jnp.max(ab, axis=1, keepdims=True)
        e = jnp.where(colmask, jnp.exp(ab), 0.0)
        acc = acc + e / jnp.sum(e, axis=1, keepdims=True)
    adj = acc * (1.0 / _B)

    work = jnp.where(colmask, adj, -1.0)
    thresh = None
    for _ in range(_TOPK):
        thresh = jnp.max(work, axis=1, keepdims=True)
        ismax = work == thresh
        pos = jnp.min(jnp.where(ismax, col, _NP), axis=1, keepdims=True)
        work = jnp.where(col == pos, -1.0, work)
    rowmask = lax.broadcasted_iota(jnp.int32, (_NP, _NP), 0) < _N
    adj_k = jnp.where((adj >= thresh) & colmask & rowmask, adj, 0.0)
    d = jnp.sum(adj_k, axis=1, keepdims=True)
    dinv = jnp.where(d > 0.0, 1.0 / d, 0.0)
    support = dinv * adj_k

    def smul(xv):
        w = xv.shape[1]
        g = min(_B, max(1, 128 // w))
        parts = [None] * _B
        for gi in range(_B // g):
            xs = [xv[(gi * g + j) * _NP:(gi * g + j + 1) * _NP, :]
                  for j in range(g)]
            blk = xs[0] if g == 1 else jnp.concatenate(xs, axis=1)
            pr = jnp.dot(support, blk, preferred_element_type=f32)
            for j in range(g):
                parts[gi * g + j] = pr[:, j * w:(j + 1) * w]
        return jnp.concatenate(parts, axis=0)

    def cell(xin, hin, wgf, bg, wcf, bc):
        cin = xin.shape[1]
        c = cin + _HID
        wgx = jnp.concatenate([wgf[m * c:m * c + cin] for m in range(_NM)], axis=0)
        wgh = jnp.concatenate([wgf[m * c + cin:(m + 1) * c] for m in range(_NM)], axis=0)
        wcx = jnp.concatenate([wcf[m * c:m * c + cin] for m in range(_NM)], axis=0)
        wch = jnp.concatenate([wcf[m * c + cin:(m + 1) * c] for m in range(_NM)], axis=0)
        sx1 = smul(xin)
        sx2 = 2.0 * smul(sx1) - xin
        sh1 = smul(hin)
        sh2 = 2.0 * smul(sh1) - hin
        xstack = jnp.concatenate([xin, sx1, sx2], axis=1)
        hstack = jnp.concatenate([hin, sh1, sh2], axis=1)
        g = sig(jnp.dot(xstack, wgx, preferred_element_type=f32)
                + jnp.dot(hstack, wgh, preferred_element_type=f32) + bg)
        r = g[:, :_HID]
        u = g[:, _HID:]
        rh = r * hin
        t1 = smul(rh)
        t2 = 2.0 * smul(t1) - rh
        tstack = jnp.concatenate([rh, t1, t2], axis=1)
        cand = jnp.tanh(jnp.dot(xstack, wcx, preferred_element_type=f32)
                        + jnp.dot(tstack, wch, preferred_element_type=f32) + bc)
        return u * hin + (1.0 - u) * cand

    e0g = e0gx_ref[...]; e0bg = e0bg_ref[...]
    e0c = e0cx_ref[...]; e0bc = e0bc_ref[...]
    e1g = e1gx_ref[...]; e1bg = e1bg_ref[...]
    e1c = e1cx_ref[...]; e1bc = e1bc_ref[...]
    d0g = d0gx_ref[...]; d0bg = d0bg_ref[...]
    d0c = d0cx_ref[...]; d0bc = d0bc_ref[...]
    d1g = d1gx_ref[...]; d1bg = d1bg_ref[...]
    d1c = d1cx_ref[...]; d1bc = d1bc_ref[...]
    projw = projw_ref[...]
    projb = projb_ref[...]

    def enc_step(t, hh):
        h0, h1 = hh
        x_t = xt_ref[pl.ds(t, 1)].reshape(_BNP, _IN)
        h0 = cell(x_t, h0, e0g, e0bg, e0c, e0bc)
        h1 = cell(h0, h1, e1g, e1bg, e1c, e1bc)
        return (h0, h1)

    zst = jnp.zeros((_BNP, _HID), f32)
    h0, h1 = lax.fori_loop(0, _T, enc_step, (zst, zst))

    tcol = lax.broadcasted_iota(jnp.int32, (_BNP, _T), 1)

    def dec_step(t, carry):
        g0, g1, cur, outacc = carry
        g0 = cell(cur, g0, d0g, d0bg, d0c, d0bc)
        g1 = cell(g0, g1, d1g, d1bg, d1c, d1bc)
        p = jnp.sum(g1 * projw, axis=1, keepdims=True) + projb
        outacc = jnp.where(tcol == t, p, outacc)
        return (g0, g1, p, outacc)

    _, _, _, outacc = lax.fori_loop(
        0, _T, dec_step,
        (h0, h1, jnp.zeros((_BNP, _OUT), f32), jnp.zeros((_BNP, _T), f32)))
    out_ref[...] = outacc


def kernel(x, enc0_Wg, enc0_bg, enc0_Wc, enc0_bc, enc1_Wg, enc1_bg, enc1_Wc, enc1_bc,
           dec0_Wg, dec0_bg, dec0_Wc, dec0_bc, dec1_Wg, dec1_bg, dec1_Wc, dec1_bc,
           proj_W, proj_b, gru_Wih, gru_Whh, gru_bih, gru_bhh, Wkey, Wquery):
    f32 = jnp.float32
    xt = jnp.transpose(x, (1, 0, 2, 3))
    xt = jnp.pad(xt, ((0, 0), (0, 0), (0, _NP - _N), (0, 0)))
    xt = xt.reshape(_T, _BNP, _IN)

    def deint(w):
        return jnp.concatenate([w[m::_NM] for m in range(_NM)], axis=0)

    args = (
        xt,
        gru_Wih.T.astype(f32), gru_Whh.T.astype(f32),
        gru_bih[None], gru_bhh[None],
        Wkey, Wquery,
        deint(enc0_Wg), enc0_bg[None], deint(enc0_Wc), enc0_bc[None],
        deint(enc1_Wg), enc1_bg[None], deint(enc1_Wc), enc1_bc[None],
        deint(dec0_Wg), dec0_bg[None], deint(dec0_Wc), dec0_bc[None],
        deint(dec1_Wg), dec1_bg[None], deint(dec1_Wc), dec1_bc[None],
        proj_W.T, proj_b.reshape(1, 1),
    )

    out = pl.pallas_call(
        _body,
        out_shape=jax.ShapeDtypeStruct((_BNP, _T), f32),
    )(*args)
    out = out.reshape(_B, _NP, _T)
    return jnp.transpose(out, (0, 2, 1))[:, :, :_N, None]
